# baseline (speedup 1.0000x reference)
import functools

import jax
import jax.numpy as jnp
from jax import lax
from jax.experimental import pallas as pl
from jax.experimental.pallas import tpu as pltpu
from jax.experimental.pallas import tpu_sc as plsc

NC = 2
NS = 16
NW = NC * NS
LANES = 16


def _tree_sum(vals):
    while len(vals) > 1:
        nxt = [vals[i] + vals[i + 1] for i in range(0, len(vals) - 1, 2)]
        if len(vals) % 2:
            nxt.append(vals[-1])
        vals = nxt
    return vals[0]


def _make_pool_kernel(B, L, LP, D, V):
    BPW = B // NW
    mesh = plsc.VectorSubcoreMesh(core_axis_name="c", subcore_axis_name="s")

    @functools.partial(
        pl.kernel,
        mesh=mesh,
        compiler_params=pltpu.CompilerParams(use_tc_tiling_on_sc=False),
        out_type=jax.ShapeDtypeStruct((B, D), jnp.float32),
        scratch_types=[
            pltpu.VMEM((BPW, LP), jnp.int32),
            pltpu.VMEM((LP, D), jnp.float32),
            pltpu.VMEM((LP, D), jnp.float32),
            pltpu.VMEM((BPW, D), jnp.float32),
            pltpu.SemaphoreType.DMA,
            pltpu.SemaphoreType.DMA,
        ],
    )
    def pool(ids_hbm, table_hbm, out_hbm, idx_v, rows0, rows1, out_v, sem0, sem1):
        wid = lax.axis_index("s") * NC + lax.axis_index("c")
        base = wid * BPW
        pltpu.sync_copy(ids_hbm.at[pl.ds(base, BPW)], idx_v)

        pltpu.async_copy(table_hbm.at[idx_v.at[0]], rows0, sem0)
        pltpu.async_copy(table_hbm.at[idx_v.at[1]], rows1, sem1)

        def reduce_into(rows, i):
            for j in range(D // LANES):
                sl = pl.ds(j * LANES, LANES)
                out_v[i, sl] = _tree_sum([rows[r, sl] for r in range(L)])

        @pl.loop(0, BPW, step=2)
        def _(i):
            pltpu.make_async_copy(table_hbm.at[idx_v.at[0]], rows0, sem0).wait()
            reduce_into(rows0, i)

            @pl.when(i + 2 < BPW)
            def _():
                pltpu.async_copy(table_hbm.at[idx_v.at[i + 2]], rows0, sem0)

            pltpu.make_async_copy(table_hbm.at[idx_v.at[1]], rows1, sem1).wait()
            reduce_into(rows1, i + 1)

            @pl.when(i + 3 < BPW)
            def _():
                pltpu.async_copy(table_hbm.at[idx_v.at[i + 3]], rows1, sem1)

        pltpu.sync_copy(out_v, out_hbm.at[pl.ds(base, BPW)])

    return pool


def _project(pooled, Ws, b2d):
    B, D = pooled.shape
    T = Ws.shape[1]

    def body(x_ref, w_ref, b_ref, o_ref):
        o_ref[...] = (
            jnp.dot(x_ref[...], w_ref[...], preferred_element_type=jnp.float32)
            + b_ref[...]
        )

    return pl.pallas_call(
        body,
        out_shape=jax.ShapeDtypeStruct((B, T), jnp.float32),
    )(pooled, Ws, b2d)


@jax.jit
def kernel(text_ids, table, W, b):
    B, L = text_ids.shape
    V, D = table.shape
    T = W.shape[1]
    LP = (L + 7) // 8 * 8
    ids_p = jnp.pad(text_ids.astype(jnp.int32), ((0, 0), (0, LP - L)))
    pooled = _make_pool_kernel(B, L, LP, D, V)(ids_p, table)
    Ws = W * (1.0 / L)
    return _project(pooled, Ws, b.reshape(1, T))


# speedup vs baseline: 1.3825x; 1.3825x over previous
import functools

import jax
import jax.numpy as jnp
from jax import lax
from jax.experimental import pallas as pl
from jax.experimental.pallas import tpu as pltpu
from jax.experimental.pallas import tpu_sc as plsc

NC = 2
NS = 16
NW = NC * NS
LANES = 16
EPS = 2
NBUF = 8


def _tree_sum(vals):
    while len(vals) > 1:
        nxt = [vals[i] + vals[i + 1] for i in range(0, len(vals) - 1, 2)]
        if len(vals) % 2:
            nxt.append(vals[-1])
        vals = nxt
    return vals[0]


def _make_pool_kernel(B, L, D, V):
    LW = (EPS * L + 7) // 8 * 8
    NR = B // EPS
    SPW = NR // NW
    BPW = B // NW
    mesh = plsc.VectorSubcoreMesh(core_axis_name="c", subcore_axis_name="s")

    @functools.partial(
        pl.kernel,
        mesh=mesh,
        compiler_params=pltpu.CompilerParams(use_tc_tiling_on_sc=False),
        out_type=jax.ShapeDtypeStruct((B, D), jnp.float32),
        scratch_types=[
            pltpu.VMEM((SPW, LW), jnp.int32),
            [pltpu.VMEM((LW, D), jnp.float32)] * NBUF,
            pltpu.VMEM((BPW, D), jnp.float32),
            [pltpu.SemaphoreType.DMA] * NBUF,
        ],
    )
    def pool(ids_hbm, table_hbm, out_hbm, idx_v, bufs, out_v, sems):
        wid = lax.axis_index("s") * NC + lax.axis_index("c")
        base = wid * SPW
        pltpu.sync_copy(ids_hbm.at[pl.ds(base, SPW)], idx_v)

        def issue(s, k):
            pltpu.async_copy(table_hbm.at[idx_v.at[s]], bufs[k], sems[k])

        for k in range(NBUF):
            issue(k, k)

        def process(rows, s):
            @pl.loop(0, EPS)
            def _(e):
                for j in range(D // LANES):
                    sl = pl.ds(j * LANES, LANES)
                    out_v[EPS * s + e, sl] = _tree_sum(
                        [rows[e * L + r, sl] for r in range(L)]
                    )

        @pl.loop(0, SPW, step=NBUF)
        def _(s):
            for k in range(NBUF):
                pltpu.make_async_copy(
                    table_hbm.at[idx_v.at[0]], bufs[k], sems[k]
                ).wait()
                process(bufs[k], s + k)

                @pl.when(s + NBUF + k < SPW)
                def _():
                    issue(s + NBUF + k, k)

        pltpu.sync_copy(out_v, out_hbm.at[pl.ds(wid * BPW, BPW)])

    return pool


def _project(pooled, Ws, b2d):
    B, D = pooled.shape
    T = Ws.shape[1]

    def body(x_ref, w_ref, b_ref, o_ref):
        o_ref[...] = (
            jnp.dot(x_ref[...], w_ref[...], preferred_element_type=jnp.float32)
            + b_ref[...]
        )

    return pl.pallas_call(
        body,
        out_shape=jax.ShapeDtypeStruct((B, T), jnp.float32),
    )(pooled, Ws, b2d)


@jax.jit
def kernel(text_ids, table, W, b):
    B, L = text_ids.shape
    V, D = table.shape
    T = W.shape[1]
    LW = (EPS * L + 7) // 8 * 8
    ids2 = jnp.pad(
        text_ids.astype(jnp.int32).reshape(B // EPS, EPS * L),
        ((0, 0), (0, LW - EPS * L)),
    )
    pooled = _make_pool_kernel(B, L, D, V)(ids2, table)
    Ws = W * (1.0 / L)
    return _project(pooled, Ws, b.reshape(1, T))


# speedup vs baseline: 1.7221x; 1.2456x over previous
import functools

import jax
import jax.numpy as jnp
from jax import lax
from jax.experimental import pallas as pl
from jax.experimental.pallas import tpu as pltpu
from jax.experimental.pallas import tpu_sc as plsc

NC = 2
NS = 16
NW = NC * NS
LANES = 16
EPR = 2
RPS = 4
NBUF = 4


def _tree_sum(vals):
    while len(vals) > 1:
        nxt = [vals[i] + vals[i + 1] for i in range(0, len(vals) - 1, 2)]
        if len(vals) % 2:
            nxt.append(vals[-1])
        vals = nxt
    return vals[0]


def _make_pool_kernel(B, L, D, V):
    LW = EPR * RPS * L
    NR = B // (EPR * RPS)
    SPW = NR // NW
    EPS = EPR * RPS
    BPW = B // NW
    mesh = plsc.VectorSubcoreMesh(core_axis_name="c", subcore_axis_name="s")

    @functools.partial(
        pl.kernel,
        mesh=mesh,
        compiler_params=pltpu.CompilerParams(use_tc_tiling_on_sc=False),
        out_type=jax.ShapeDtypeStruct((B, D), jnp.float32),
        scratch_types=[
            pltpu.VMEM((SPW, LW), jnp.int32),
            [pltpu.VMEM((LW, D), jnp.float32)] * NBUF,
            pltpu.VMEM((BPW, D), jnp.float32),
            [pltpu.SemaphoreType.DMA] * NBUF,
        ],
    )
    def pool(ids_hbm, table_hbm, out_hbm, idx_v, bufs, out_v, sems):
        wid = lax.axis_index("s") * NC + lax.axis_index("c")
        base = wid * SPW
        pltpu.sync_copy(ids_hbm.at[pl.ds(base, SPW)], idx_v)

        def issue(s, k):
            pltpu.async_copy(table_hbm.at[idx_v.at[s]], bufs[k], sems[k])

        for k in range(NBUF):
            issue(k, k)

        def process(rows, s):
            @pl.loop(0, EPS)
            def _(e):
                for j in range(D // LANES):
                    sl = pl.ds(j * LANES, LANES)
                    out_v[EPS * s + e, sl] = _tree_sum(
                        [rows[e * L + r, sl] for r in range(L)]
                    )

        @pl.loop(0, SPW, step=NBUF)
        def _(s):
            for k in range(NBUF):
                pltpu.make_async_copy(
                    table_hbm.at[idx_v.at[0]], bufs[k], sems[k]
                ).wait()
                process(bufs[k], s + k)

                @pl.when(s + NBUF + k < SPW)
                def _():
                    issue(s + NBUF + k, k)

        pltpu.sync_copy(out_v, out_hbm.at[pl.ds(wid * BPW, BPW)])

    return pool


def _project(pooled, Ws, b2d):
    B, D = pooled.shape
    T = Ws.shape[1]

    def body(x_ref, w_ref, b_ref, o_ref):
        o_ref[...] = (
            jnp.dot(x_ref[...], w_ref[...], preferred_element_type=jnp.float32)
            + b_ref[...]
        )

    return pl.pallas_call(
        body,
        out_shape=jax.ShapeDtypeStruct((B, T), jnp.float32),
    )(pooled, Ws, b2d)


@jax.jit
def kernel(text_ids, table, W, b):
    B, L = text_ids.shape
    V, D = table.shape
    T = W.shape[1]
    ids2 = text_ids.astype(jnp.int32).reshape(B // (EPR * RPS), EPR * RPS * L)
    pooled = _make_pool_kernel(B, L, D, V)(ids2, table)
    Ws = W * (1.0 / L)
    return _project(pooled, Ws, b.reshape(1, T))
